# baseline (speedup 1.0000x reference)
import functools

import jax
import jax.numpy as jnp
from jax import lax
from jax.experimental import pallas as pl
from jax.experimental.pallas import tpu as pltpu
from jax.experimental.pallas import tpu_sc as plsc

T, DI, DO, E = 8192, 2048, 2048, 8
BT = 256
NPAD = T + E * BT
NB = NPAD // BT

NW = 32
CH = 16


def _make_sc_row_gather(n_rows, row_shape, dtype):
    mpw = n_rows // NW
    nch = mpw // CH
    mesh = plsc.VectorSubcoreMesh(core_axis_name="c", subcore_axis_name="s")

    nbuf = 3

    @functools.partial(
        pl.kernel, mesh=mesh,
        out_type=jax.ShapeDtypeStruct((n_rows,) + row_shape, dtype),
        scratch_types=(
            [pltpu.VMEM((nch, CH), jnp.int32)]
            + [pltpu.VMEM((CH,) + row_shape, dtype) for _ in range(nbuf)]
            + [pltpu.SemaphoreType.DMA for _ in range(2 * nbuf)]
        ),
    )
    def gather_k(table_hbm, idx_hbm, out_hbm, idx_v, *rest):
        bufs = rest[:nbuf]
        gsems = rest[nbuf:2 * nbuf]
        wsems = rest[2 * nbuf:]
        wid = lax.axis_index("s") * 2 + lax.axis_index("c")
        base = wid * mpw
        pltpu.sync_copy(idx_hbm.at[wid], idx_v)

        gh = [None] * nch
        wh = [None] * nch
        for c in range(min(2, nch)):
            gh[c] = pltpu.async_copy(table_hbm.at[idx_v.at[c]],
                                     bufs[c % nbuf], gsems[c % nbuf])
        for c in range(nch):
            slot = c % nbuf
            gh[c].wait()
            wh[c] = pltpu.async_copy(bufs[slot],
                                     out_hbm.at[pl.ds(base + c * CH, CH)],
                                     wsems[slot])
            nxt = c + 2
            if nxt < nch:
                nslot = nxt % nbuf
                if nxt - nbuf >= 0:
                    wh[nxt - nbuf].wait()
                gh[nxt] = pltpu.async_copy(table_hbm.at[idx_v.at[nxt]],
                                           bufs[nslot], gsems[nslot])
        for c in range(max(0, nch - nbuf), nch):
            if wh[c] is not None:
                wh[c].wait()

    def run(table, idx):
        return gather_k(table, idx.reshape(NW, nch, CH))

    return run


_sc_gather_npad = _make_sc_row_gather(NPAD, (DI,), jnp.float32)
_sc_gather_t = _make_sc_row_gather(T, (DO,), jnp.float32)


def _routing(ind):
    i32 = jnp.int32
    oh = (ind[:, None] == jnp.arange(E, dtype=i32)[None, :]).astype(i32)
    counts = jnp.sum(oh, axis=0)
    padded = (counts + BT - 1) // BT * BT
    cpe = jnp.cumsum(padded)
    poff = cpe - padded
    pos = jnp.sum(oh * (jnp.cumsum(oh, 0) - 1 + poff[None, :]), axis=1)
    src = (jnp.arange(NPAD, dtype=i32) % T).at[pos].set(jnp.arange(T, dtype=i32))
    blk_start = jnp.arange(NB, dtype=i32) * BT
    block_expert = jnp.sum(blk_start[:, None] >= cpe[None, :], axis=1)
    block_expert = jnp.minimum(block_expert, E - 1).astype(i32)
    nb_real = (cpe[-1] // BT).astype(i32).reshape(1)
    return pos, src, block_expert, nb_real


def _gemm_body(be_ref, nbr_ref, xs_ref, w_ref, b_ref, ys_ref):
    @pl.when(pl.program_id(0) < nbr_ref[0])
    def _():
        acc = jax.lax.dot_general(xs_ref[...].astype(jnp.bfloat16), w_ref[0],
                                  (((1,), (1,)), ((), ())),
                                  preferred_element_type=jnp.float32)
        ys_ref[...] = acc + b_ref[0]


def kernel(x, ind, W, b):
    pos, src, block_expert, nb_real = _routing(ind)
    w16 = W.astype(jnp.bfloat16)
    xs = _sc_gather_npad(x, src)
    b3 = b.reshape(E, 1, DO)

    grid_spec = pltpu.PrefetchScalarGridSpec(
        num_scalar_prefetch=2,
        grid=(NB,),
        in_specs=[
            pl.BlockSpec((BT, DI), lambda g, be, nbr: (g, 0)),
            pl.BlockSpec((1, DO, DI), lambda g, be, nbr: (be[g], 0, 0)),
            pl.BlockSpec((1, 1, DO), lambda g, be, nbr: (be[g], 0, 0)),
        ],
        out_specs=pl.BlockSpec((BT, DO), lambda g, be, nbr: (g, 0)),
    )
    ys = pl.pallas_call(
        _gemm_body,
        grid_spec=grid_spec,
        out_shape=jax.ShapeDtypeStruct((NPAD, DO), jnp.float32),
    )(block_expert, nb_real, xs, w16, b3)
    return _sc_gather_t(ys, pos)


# speedup vs baseline: 1.1488x; 1.1488x over previous
import functools

import jax
import jax.numpy as jnp
from jax import lax
from jax.experimental import pallas as pl
from jax.experimental.pallas import tpu as pltpu
from jax.experimental.pallas import tpu_sc as plsc

T, DI, DO, E = 8192, 2048, 2048, 8
BT = 256
NPAD = T + E * BT
NB = NPAD // BT

NW = 32
CH = 16


def _make_sc_row_gather(n_rows, row_shape, dtype):
    mpw = n_rows // NW
    nch = mpw // CH
    mesh = plsc.VectorSubcoreMesh(core_axis_name="c", subcore_axis_name="s")

    nbuf = 3

    @functools.partial(
        pl.kernel, mesh=mesh,
        out_type=jax.ShapeDtypeStruct((n_rows,) + row_shape, dtype),
        scratch_types=(
            [pltpu.VMEM((nch, CH), jnp.int32)]
            + [pltpu.VMEM((CH,) + row_shape, dtype) for _ in range(nbuf)]
            + [pltpu.SemaphoreType.DMA for _ in range(2 * nbuf)]
        ),
    )
    def gather_k(table_hbm, idx_hbm, out_hbm, idx_v, *rest):
        bufs = rest[:nbuf]
        gsems = rest[nbuf:2 * nbuf]
        wsems = rest[2 * nbuf:]
        wid = lax.axis_index("s") * 2 + lax.axis_index("c")
        base = wid * mpw
        pltpu.sync_copy(idx_hbm.at[wid], idx_v)

        gh = [None] * nch
        wh = [None] * nch
        for c in range(min(2, nch)):
            gh[c] = pltpu.async_copy(table_hbm.at[idx_v.at[c]],
                                     bufs[c % nbuf], gsems[c % nbuf])
        for c in range(nch):
            slot = c % nbuf
            gh[c].wait()
            wh[c] = pltpu.async_copy(bufs[slot],
                                     out_hbm.at[pl.ds(base + c * CH, CH)],
                                     wsems[slot])
            nxt = c + 2
            if nxt < nch:
                nslot = nxt % nbuf
                if nxt - nbuf >= 0:
                    wh[nxt - nbuf].wait()
                gh[nxt] = pltpu.async_copy(table_hbm.at[idx_v.at[nxt]],
                                           bufs[nslot], gsems[nslot])
        for c in range(max(0, nch - nbuf), nch):
            if wh[c] is not None:
                wh[c].wait()

    def run(table, idx):
        return gather_k(table, idx.reshape(NW, nch, CH))

    return run


_sc_gather_npad = _make_sc_row_gather(NPAD, (DI,), jnp.float32)
_sc_gather_t = _make_sc_row_gather(T, (DO,), jnp.float32)


def _routing(ind):
    i32 = jnp.int32
    f32 = jnp.float32
    ohf = (ind[:, None] == jnp.arange(E, dtype=i32)[None, :]).astype(f32)
    CKS = 128
    NCK = T // CKS
    oh3 = ohf.reshape(NCK, CKS, E)
    tril_in = jnp.tril(jnp.ones((CKS, CKS), f32))
    within = jnp.einsum('ij,cjf->cif', tril_in, oh3)
    chunk_tot = jnp.sum(oh3, axis=1)
    tril_ex = jnp.tril(jnp.ones((NCK, NCK), f32), k=-1)
    chunk_pref = jnp.einsum('ij,jf->if', tril_ex, chunk_tot)
    cum = (within + chunk_pref[:, None, :]).reshape(T, E)
    counts = jnp.sum(chunk_tot, axis=0).astype(i32)
    padded = (counts + BT - 1) // BT * BT
    cpe = jnp.cumsum(padded)
    poff = cpe - padded
    pos = jnp.sum(ohf * (cum - 1.0 + poff.astype(f32)[None, :]),
                  axis=1).astype(i32)
    src = (jnp.arange(NPAD, dtype=i32) % T).at[pos].set(jnp.arange(T, dtype=i32))
    blk_start = jnp.arange(NB, dtype=i32) * BT
    block_expert = jnp.sum(blk_start[:, None] >= cpe[None, :], axis=1)
    block_expert = jnp.minimum(block_expert, E - 1).astype(i32)
    nb_real = (cpe[-1] // BT).astype(i32).reshape(1)
    return pos, src, block_expert, nb_real


def _gemm_body(be_ref, nbr_ref, xs_ref, w_ref, b_ref, ys_ref):
    @pl.when(pl.program_id(0) < nbr_ref[0])
    def _():
        acc = jax.lax.dot_general(xs_ref[...], w_ref[0],
                                  (((1,), (1,)), ((), ())),
                                  preferred_element_type=jnp.float32)
        ys_ref[...] = acc + b_ref[0]


def kernel(x, ind, W, b):
    pos, src, block_expert, nb_real = _routing(ind)
    xs = _sc_gather_npad(x, src)
    b3 = b.reshape(E, 1, DO)

    grid_spec = pltpu.PrefetchScalarGridSpec(
        num_scalar_prefetch=2,
        grid=(NB,),
        in_specs=[
            pl.BlockSpec((BT, DI), lambda g, be, nbr: (g, 0)),
            pl.BlockSpec((1, DO, DI), lambda g, be, nbr: (be[g], 0, 0)),
            pl.BlockSpec((1, 1, DO), lambda g, be, nbr: (be[g], 0, 0)),
        ],
        out_specs=pl.BlockSpec((BT, DO), lambda g, be, nbr: (g, 0)),
    )
    ys = pl.pallas_call(
        _gemm_body,
        grid_spec=grid_spec,
        out_shape=jax.ShapeDtypeStruct((NPAD, DO), jnp.float32),
    )(block_expert, nb_real, xs, W, b3)
    return _sc_gather_t(ys, pos)


# speedup vs baseline: 1.1532x; 1.0038x over previous
import functools

import jax
import jax.numpy as jnp
from jax import lax
from jax.experimental import pallas as pl
from jax.experimental.pallas import tpu as pltpu
from jax.experimental.pallas import tpu_sc as plsc

T, DI, DO, E = 8192, 2048, 2048, 8
BT = 256
NPAD = T + E * BT
NB = NPAD // BT

NW = 32
CH = 16


def _make_sc_row_gather(n_rows, row_shape, dtype):
    mpw = n_rows // NW
    nch = mpw // CH
    mesh = plsc.VectorSubcoreMesh(core_axis_name="c", subcore_axis_name="s")

    nbuf = 3

    @functools.partial(
        pl.kernel, mesh=mesh,
        out_type=jax.ShapeDtypeStruct((n_rows,) + row_shape, dtype),
        scratch_types=(
            [pltpu.VMEM((nch, CH), jnp.int32)]
            + [pltpu.VMEM((CH,) + row_shape, dtype) for _ in range(nbuf)]
            + [pltpu.SemaphoreType.DMA for _ in range(2 * nbuf)]
        ),
    )
    def gather_k(table_hbm, idx_hbm, out_hbm, idx_v, *rest):
        bufs = rest[:nbuf]
        gsems = rest[nbuf:2 * nbuf]
        wsems = rest[2 * nbuf:]
        wid = lax.axis_index("s") * 2 + lax.axis_index("c")
        base = wid * mpw
        pltpu.sync_copy(idx_hbm.at[wid], idx_v)

        gh = [None] * nch
        wh = [None] * nch
        for c in range(min(2, nch)):
            gh[c] = pltpu.async_copy(table_hbm.at[idx_v.at[c]],
                                     bufs[c % nbuf], gsems[c % nbuf])
        for c in range(nch):
            slot = c % nbuf
            gh[c].wait()
            wh[c] = pltpu.async_copy(bufs[slot],
                                     out_hbm.at[pl.ds(base + c * CH, CH)],
                                     wsems[slot])
            nxt = c + 2
            if nxt < nch:
                nslot = nxt % nbuf
                if nxt - nbuf >= 0:
                    wh[nxt - nbuf].wait()
                gh[nxt] = pltpu.async_copy(table_hbm.at[idx_v.at[nxt]],
                                           bufs[nslot], gsems[nslot])
        for c in range(max(0, nch - nbuf), nch):
            if wh[c] is not None:
                wh[c].wait()

    def run(table, idx):
        return gather_k(table, idx.reshape(NW, nch, CH))

    return run


_sc_gather_t = _make_sc_row_gather(T, (DO,), jnp.float32)


def _routing(ind):
    i32 = jnp.int32
    f32 = jnp.float32
    ohf = (ind[:, None] == jnp.arange(E, dtype=i32)[None, :]).astype(f32)
    CKS = 128
    NCK = T // CKS
    oh3 = ohf.reshape(NCK, CKS, E)
    tril_in = jnp.tril(jnp.ones((CKS, CKS), f32))
    within = jnp.einsum('ij,cjf->cif', tril_in, oh3)
    chunk_tot = jnp.sum(oh3, axis=1)
    tril_ex = jnp.tril(jnp.ones((NCK, NCK), f32), k=-1)
    chunk_pref = jnp.einsum('ij,jf->if', tril_ex, chunk_tot)
    cum = (within + chunk_pref[:, None, :]).reshape(T, E)
    counts = jnp.sum(chunk_tot, axis=0).astype(i32)
    padded = (counts + BT - 1) // BT * BT
    cpe = jnp.cumsum(padded)
    poff = cpe - padded
    pos = jnp.sum(ohf * (cum - 1.0 + poff.astype(f32)[None, :]),
                  axis=1).astype(i32)
    src = (jnp.arange(NPAD, dtype=i32) % T).at[pos].set(jnp.arange(T, dtype=i32))
    blk_start = jnp.arange(NB, dtype=i32) * BT
    block_expert = jnp.sum(blk_start[:, None] >= cpe[None, :], axis=1)
    block_expert = jnp.minimum(block_expert, E - 1).astype(i32)
    nb_real = (cpe[-1] // BT).astype(i32).reshape(1)
    return pos, src, block_expert, nb_real


NBH = NB // 2
NPH = NPAD // 2


def _make_gemm_half(half):
    off = half * NBH

    def body(be_ref, nbr_ref, xs_ref, w_ref, b_ref, *rest):
        ys_ref = rest[-1]

        @pl.when(pl.program_id(0) + off < nbr_ref[0])
        def _():
            acc = jax.lax.dot_general(xs_ref[...], w_ref[0],
                                      (((1,), (1,)), ((), ())),
                                      preferred_element_type=jnp.float32)
            ys_ref[...] = acc + b_ref[0]

    in_specs = [
        pl.BlockSpec((BT, DI), lambda g, be, nbr: (g, 0)),
        pl.BlockSpec((1, DO, DI), lambda g, be, nbr: (be[g], 0, 0)),
        pl.BlockSpec((1, 1, DO), lambda g, be, nbr: (be[g], 0, 0)),
    ]
    kwargs = {}
    if half == 1:
        in_specs.append(pl.BlockSpec(memory_space=pl.MemorySpace.ANY))
        kwargs["input_output_aliases"] = {5: 0}
    grid_spec = pltpu.PrefetchScalarGridSpec(
        num_scalar_prefetch=2,
        grid=(NBH,),
        in_specs=in_specs,
        out_specs=pl.BlockSpec((BT, DO), lambda g, be, nbr: (g + off, 0)),
    )
    return pl.pallas_call(
        body, grid_spec=grid_spec,
        out_shape=jax.ShapeDtypeStruct((NPAD, DO), jnp.float32), **kwargs)


_sc_gather_half = _make_sc_row_gather(NPH, (DI,), jnp.float32)
_gemm_a = _make_gemm_half(0)
_gemm_b = _make_gemm_half(1)


def kernel(x, ind, W, b):
    pos, src, block_expert, nb_real = _routing(ind)
    b3 = b.reshape(E, 1, DO)
    xs_a = _sc_gather_half(x, src[:NPH])
    xs_b = _sc_gather_half(x, src[NPH:])
    ys_a = _gemm_a(block_expert[:NBH], nb_real, xs_a, W, b3)
    ys = _gemm_b(block_expert[NBH:], nb_real, xs_b, W, b3, ys_a)
    return _sc_gather_t(ys, pos)
